# baseline (speedup 1.0000x reference)
import functools

import jax
import jax.numpy as jnp
from jax import lax
from jax.experimental import pallas as pl
from jax.experimental.pallas import tpu as pltpu
from jax.experimental.pallas import tpu_sc as plsc

_PRIMES = (179424941, 179425457)
_M = 200000
_D = 64

_NC, _NS, _L = 2, 16, 16
_NW = _NC * _NS

_P0 = _PRIMES[0] % _M
_P1 = _PRIMES[1] % _M
_C0 = (1024 * _PRIMES[0]) % _M
_C1 = (1024 * _PRIMES[1]) % _M

_NB = 4
_CH = 200
_GS = (80, 80, 40)


@functools.partial(jax.jit, static_argnums=(2, 3))
def _sc_lookup(idx2d, table_pad, b, s):
    b_per_w = b // _NW
    n_chunk = b_per_w // _NB
    mesh = plsc.VectorSubcoreMesh(core_axis_name="c", subcore_axis_name="s")

    @functools.partial(
        pl.kernel,
        out_type=jax.ShapeDtypeStruct((b, s, _D), jnp.float32),
        mesh=mesh,
        scratch_types=[
            pltpu.VMEM((_NB, s), jnp.int32),
            pltpu.VMEM((_CH,), jnp.int32),
            pltpu.VMEM((3, 80), jnp.int32),
            pltpu.VMEM((3, 80), jnp.int32),
            pltpu.VMEM((_CH, 2 * _D), jnp.float32),
            pltpu.VMEM((_CH, 2 * _D), jnp.float32),
            pltpu.VMEM((_NB, s, _D), jnp.float32),
            pltpu.SemaphoreType.DMA,
        ],
    )
    def k(idx_hbm, table_hbm, out_hbm, idx_v, f_v, h0_v, h1_v, r0_v, r1_v,
          o_v, sem):
        wid = lax.axis_index("s") * jnp.int32(_NC) + lax.axis_index("c")
        base = wid * jnp.int32(b_per_w)

        @pl.loop(jnp.int32(0), jnp.int32(n_chunk))
        def _(g):
            b0 = base + g * jnp.int32(_NB)
            pltpu.sync_copy(idx_hbm.at[pl.ds(b0, _NB)], idx_v)

            for t in range(_NB):
                for src in (0, 16, 32, 34):
                    f_v[pl.ds(jnp.int32(50 * t + src), _L)] = (
                        idx_v[jnp.int32(t), pl.ds(jnp.int32(src), _L)])

            for gi in range(13):
                p = 184 if gi == 12 else 16 * gi
                row, col = p // 80, p % 80
                v = f_v[pl.ds(jnp.int32(p), _L)]
                hi = lax.shift_right_logical(v, jnp.int32(10))
                lo = lax.bitwise_and(v, jnp.int32(1023))
                m = jnp.int32(_M)
                h0_v[row, pl.ds(jnp.int32(col), _L)] = lax.rem(
                    hi * jnp.int32(_C0) + lo * jnp.int32(_P0), m)
                h1_v[row, pl.ds(jnp.int32(col), _L)] = lax.rem(
                    hi * jnp.int32(_C1) + lo * jnp.int32(_P1), m)

            copies = []
            off = 0
            for a, gw in enumerate(_GS):
                i0 = h0_v.at[jnp.int32(a)]
                i1 = h1_v.at[jnp.int32(a)]
                if gw != 80:
                    i0 = i0.at[pl.ds(jnp.int32(0), gw)]
                    i1 = i1.at[pl.ds(jnp.int32(0), gw)]
                copies.append(pltpu.async_copy(
                    table_hbm.at[i0], r0_v.at[pl.ds(jnp.int32(off), gw)], sem))
                copies.append(pltpu.async_copy(
                    table_hbm.at[i1], r1_v.at[pl.ds(jnp.int32(off), gw)], sem))
                off += gw
            for cp in copies:
                cp.wait()

            for t in range(_NB):
                @pl.loop(jnp.int32(0), jnp.int32(s), step=jnp.int32(2))
                def _(si, t=t):
                    for dsi in range(2):
                        srow = si + jnp.int32(dsi)
                        p = jnp.int32(50 * t) + srow
                        for c in range(0, _D, _L):
                            o_v[jnp.int32(t), srow, pl.ds(jnp.int32(c), _L)] = (
                                r0_v[p, pl.ds(jnp.int32(c), _L)]
                                + r1_v[p, pl.ds(jnp.int32(c), _L)]
                            )

            ocopies = []
            for t in range(_NB):
                ocopies.append(pltpu.async_copy(
                    o_v.at[jnp.int32(t)], out_hbm.at[b0 + jnp.int32(t)], sem))
            for cp in ocopies:
                cp.wait()

    return k(idx2d, table_pad)


def kernel(indices, table):
    b, s = indices.shape
    table_pad = jnp.pad(table, ((0, 0), (0, 128 - _D)))
    out = _sc_lookup(indices.astype(jnp.int32), table_pad, b, s)
    return out


# speedup vs baseline: 1.3236x; 1.3236x over previous
import functools

import jax
import jax.numpy as jnp
from jax import lax
from jax.experimental import pallas as pl
from jax.experimental.pallas import tpu as pltpu
from jax.experimental.pallas import tpu_sc as plsc

_PRIMES = (179424941, 179425457)
_M = 200000
_D = 64

_NC, _NS, _L = 2, 16, 16
_NW = _NC * _NS

_P0 = _PRIMES[0] % _M
_P1 = _PRIMES[1] % _M
_C0 = (1024 * _PRIMES[0]) % _M
_C1 = (1024 * _PRIMES[1]) % _M

_GW = 128
_KG = 2
_CHUNK = _GW * _KG


@functools.partial(jax.jit, static_argnums=(2,))
def _sc_lookup(idx128, table, n):
    per_w = n // _NW
    n_chunk = per_w // _CHUNK
    rows_per_chunk = _CHUNK // _GW
    mesh = plsc.VectorSubcoreMesh(core_axis_name="c", subcore_axis_name="s")

    @functools.partial(
        pl.kernel,
        out_type=jax.ShapeDtypeStruct((n // 2, 2 * _D), jnp.float32),
        mesh=mesh,
        compiler_params=pltpu.CompilerParams(use_tc_tiling_on_sc=False),
        scratch_types=[
            pltpu.VMEM((_KG, _GW), jnp.int32),
            pltpu.VMEM((_KG, _GW), jnp.int32),
            pltpu.VMEM((_KG, _GW), jnp.int32),
            pltpu.VMEM((_CHUNK, _D), jnp.float32),
            pltpu.VMEM((_CHUNK, _D), jnp.float32),
            pltpu.VMEM((_CHUNK // 2, 2 * _D), jnp.float32),
            pltpu.SemaphoreType.DMA,
        ],
    )
    def k(idx_hbm, table_hbm, out_hbm, idx_v, h0_v, h1_v, r0_v, r1_v, o_v,
          sem):
        wid = lax.axis_index("s") * jnp.int32(_NC) + lax.axis_index("c")
        base = wid * jnp.int32(per_w // _GW)

        @pl.loop(jnp.int32(0), jnp.int32(n_chunk))
        def _(g):
            row_off = base + g * jnp.int32(rows_per_chunk)
            pltpu.sync_copy(idx_hbm.at[pl.ds(row_off, rows_per_chunk)], idx_v)

            for a in range(_KG):
                @pl.loop(jnp.int32(0), jnp.int32(_GW), step=jnp.int32(_L))
                def _(j, a=a):
                    v = idx_v[jnp.int32(a), pl.ds(j, _L)]
                    hi = lax.shift_right_logical(v, jnp.int32(10))
                    lo = lax.bitwise_and(v, jnp.int32(1023))
                    m = jnp.int32(_M)
                    h0_v[a, pl.ds(j, _L)] = lax.rem(
                        hi * jnp.int32(_C0) + lo * jnp.int32(_P0), m)
                    h1_v[a, pl.ds(j, _L)] = lax.rem(
                        hi * jnp.int32(_C1) + lo * jnp.int32(_P1), m)

            copies = []
            for a in range(_KG):
                copies.append(pltpu.async_copy(
                    table_hbm.at[h0_v.at[jnp.int32(a)]],
                    r0_v.at[pl.ds(jnp.int32(a * _GW), _GW)], sem))
                copies.append(pltpu.async_copy(
                    table_hbm.at[h1_v.at[jnp.int32(a)]],
                    r1_v.at[pl.ds(jnp.int32(a * _GW), _GW)], sem))
            for cp in copies:
                cp.wait()

            @pl.loop(jnp.int32(0), jnp.int32(_CHUNK // 2), step=jnp.int32(4))
            def _(i):
                for r in range(4):
                    orow = i + jnp.int32(r)
                    for half in range(2):
                        p = orow * jnp.int32(2) + jnp.int32(half)
                        for c in range(0, _D, _L):
                            o_v[orow, pl.ds(jnp.int32(half * _D + c), _L)] = (
                                r0_v[p, pl.ds(jnp.int32(c), _L)]
                                + r1_v[p, pl.ds(jnp.int32(c), _L)]
                            )

            out_off = (base + g * jnp.int32(rows_per_chunk)) * jnp.int32(
                _GW // 2)
            pltpu.sync_copy(o_v, out_hbm.at[pl.ds(out_off, _CHUNK // 2)])

    return k(idx128, table)


def kernel(indices, table):
    b, s = indices.shape
    n = b * s
    idx128 = indices.astype(jnp.int32).reshape(n // 128, 128)
    out = _sc_lookup(idx128, table, n)
    return out.reshape(b, s, _D)


# speedup vs baseline: 1.6206x; 1.2244x over previous
import functools

import jax
import jax.numpy as jnp
from jax import lax
from jax.experimental import pallas as pl
from jax.experimental.pallas import tpu as pltpu
from jax.experimental.pallas import tpu_sc as plsc

_PRIMES = (179424941, 179425457)
_M = 200000
_D = 64

_NC, _NS, _L = 2, 16, 16
_NW = _NC * _NS

_P0 = _PRIMES[0] % _M
_P1 = _PRIMES[1] % _M
_C0 = (1024 * _PRIMES[0]) % _M
_C1 = (1024 * _PRIMES[1]) % _M

_GW = 128
_KG = 2
_CHUNK = _GW * _KG


@functools.partial(jax.jit, static_argnums=(2,))
def _sc_lookup(idx128, table, n):
    per_w = n // _NW
    n_chunk = per_w // _CHUNK
    pairs = n_chunk // 2
    rpc = _CHUNK // _GW
    mesh = plsc.VectorSubcoreMesh(core_axis_name="c", subcore_axis_name="s")

    buf = lambda: [
        pltpu.VMEM((_KG, _GW), jnp.int32),
        pltpu.VMEM((_KG, _GW), jnp.int32),
        pltpu.VMEM((_KG, _GW), jnp.int32),
        pltpu.VMEM((_CHUNK, _D), jnp.float32),
        pltpu.VMEM((_CHUNK, _D), jnp.float32),
        pltpu.VMEM((_CHUNK, _D), jnp.float32),
        pltpu.SemaphoreType.DMA,
        pltpu.SemaphoreType.DMA,
    ]

    @functools.partial(
        pl.kernel,
        out_type=jax.ShapeDtypeStruct((n, _D), jnp.float32),
        mesh=mesh,
        compiler_params=pltpu.CompilerParams(use_tc_tiling_on_sc=False),
        scratch_types=buf() + buf(),
    )
    def k(idx_hbm, table_hbm, out_hbm,
          idx_a, h0_a, h1_a, r0_a, r1_a, o_a, sga, soa,
          idx_b, h0_b, h1_b, r0_b, r1_b, o_b, sgb, sob):
        wid = lax.axis_index("s") * jnp.int32(_NC) + lax.axis_index("c")
        base = wid * jnp.int32(per_w // _GW)

        def fire(g, idx_v, h0_v, h1_v, r0_v, r1_v, sg):
            row_off = base + g * jnp.int32(rpc)
            pltpu.sync_copy(idx_hbm.at[pl.ds(row_off, rpc)], idx_v)
            for a in range(_KG):
                @pl.loop(jnp.int32(0), jnp.int32(_GW), step=jnp.int32(_L))
                def _(j, a=a):
                    v = idx_v[jnp.int32(a), pl.ds(j, _L)]
                    hi = lax.shift_right_logical(v, jnp.int32(10))
                    lo = lax.bitwise_and(v, jnp.int32(1023))
                    m = jnp.int32(_M)
                    h0_v[a, pl.ds(j, _L)] = lax.rem(
                        hi * jnp.int32(_C0) + lo * jnp.int32(_P0), m)
                    h1_v[a, pl.ds(j, _L)] = lax.rem(
                        hi * jnp.int32(_C1) + lo * jnp.int32(_P1), m)
            for a in range(_KG):
                pltpu.async_copy(
                    table_hbm.at[h0_v.at[jnp.int32(a)]],
                    r0_v.at[pl.ds(jnp.int32(a * _GW), _GW)], sg)
                pltpu.async_copy(
                    table_hbm.at[h1_v.at[jnp.int32(a)]],
                    r1_v.at[pl.ds(jnp.int32(a * _GW), _GW)], sg)

        def drain(h0_v, h1_v, r0_v, r1_v, sg):
            for a in range(_KG):
                pltpu.make_async_copy(
                    table_hbm.at[h0_v.at[jnp.int32(a)]],
                    r0_v.at[pl.ds(jnp.int32(a * _GW), _GW)], sg).wait()
                pltpu.make_async_copy(
                    table_hbm.at[h1_v.at[jnp.int32(a)]],
                    r1_v.at[pl.ds(jnp.int32(a * _GW), _GW)], sg).wait()

        def wait_out(o_v, so):
            pltpu.make_async_copy(
                o_v, out_hbm.at[pl.ds(jnp.int32(0), _CHUNK)], so).wait()

        def add_store(g, r0_v, r1_v, o_v, so):
            @pl.loop(jnp.int32(0), jnp.int32(_CHUNK), step=jnp.int32(8))
            def _(i):
                for r in range(8):
                    row = i + jnp.int32(r)
                    for c in range(0, _D, _L):
                        o_v[row, pl.ds(jnp.int32(c), _L)] = (
                            r0_v[row, pl.ds(jnp.int32(c), _L)]
                            + r1_v[row, pl.ds(jnp.int32(c), _L)]
                        )
            out_off = (base + g * jnp.int32(rpc)) * jnp.int32(_GW)
            pltpu.async_copy(o_v, out_hbm.at[pl.ds(out_off, _CHUNK)], so)

        fire(jnp.int32(0), idx_a, h0_a, h1_a, r0_a, r1_a, sga)

        @pl.loop(jnp.int32(0), jnp.int32(pairs))
        def _(p):
            g0 = p * jnp.int32(2)
            fire(g0 + jnp.int32(1), idx_b, h0_b, h1_b, r0_b, r1_b, sgb)

            drain(h0_a, h1_a, r0_a, r1_a, sga)

            @pl.when(p > jnp.int32(0))
            def _():
                wait_out(o_a, soa)
            add_store(g0, r0_a, r1_a, o_a, soa)

            @pl.when(p < jnp.int32(pairs - 1))
            def _():
                fire(g0 + jnp.int32(2), idx_a, h0_a, h1_a, r0_a, r1_a, sga)

            drain(h0_b, h1_b, r0_b, r1_b, sgb)

            @pl.when(p > jnp.int32(0))
            def _():
                wait_out(o_b, sob)
            add_store(g0 + jnp.int32(1), r0_b, r1_b, o_b, sob)

        wait_out(o_a, soa)
        wait_out(o_b, sob)

    return k(idx128, table)


def kernel(indices, table):
    b, s = indices.shape
    n = b * s
    idx128 = indices.astype(jnp.int32).reshape(n // 128, 128)
    out = _sc_lookup(idx128, table, n)
    return out.reshape(b, s, _D)


# speedup vs baseline: 1.6520x; 1.0194x over previous
import functools

import jax
import jax.numpy as jnp
from jax import lax
from jax.experimental import pallas as pl
from jax.experimental.pallas import tpu as pltpu
from jax.experimental.pallas import tpu_sc as plsc

_PRIMES = (179424941, 179425457)
_M = 200000
_D = 64

_NC, _NS, _L = 2, 16, 16
_NW = _NC * _NS

_P0 = _PRIMES[0] % _M
_P1 = _PRIMES[1] % _M
_C0 = (1024 * _PRIMES[0]) % _M
_C1 = (1024 * _PRIMES[1]) % _M

_GW = 128
_KG = 2
_CHUNK = _GW * _KG


@functools.partial(jax.jit, static_argnums=(2,))
def _sc_lookup(idx128, table, n):
    per_w = n // _NW
    n_chunk = per_w // _CHUNK
    pairs = n_chunk // 2
    rpc = _CHUNK // _GW
    mesh = plsc.VectorSubcoreMesh(core_axis_name="c", subcore_axis_name="s")

    buf = lambda: [
        pltpu.VMEM((_KG, _GW), jnp.int32),
        pltpu.VMEM((_KG, _GW), jnp.int32),
        pltpu.VMEM((_KG, _GW), jnp.int32),
        pltpu.VMEM((_CHUNK, _D), jnp.float32),
        pltpu.VMEM((_CHUNK, _D), jnp.float32),
        pltpu.VMEM((_CHUNK, _D), jnp.float32),
        pltpu.SemaphoreType.DMA,
        pltpu.SemaphoreType.DMA,
    ]

    @functools.partial(
        pl.kernel,
        out_type=jax.ShapeDtypeStruct((n, _D), jnp.float32),
        mesh=mesh,
        compiler_params=pltpu.CompilerParams(use_tc_tiling_on_sc=False),
        scratch_types=buf() + buf(),
    )
    def k(idx_hbm, table_hbm, out_hbm,
          idx_a, h0_a, h1_a, r0_a, r1_a, o_a, sga, soa,
          idx_b, h0_b, h1_b, r0_b, r1_b, o_b, sgb, sob):
        wid = lax.axis_index("s") * jnp.int32(_NC) + lax.axis_index("c")
        base = wid * jnp.int32(per_w // _GW)

        def fire(g, idx_v, h0_v, h1_v, r0_v, r1_v, sg):
            row_off = base + g * jnp.int32(rpc)
            pltpu.sync_copy(idx_hbm.at[pl.ds(row_off, rpc)], idx_v)
            for a in range(_KG):
                @pl.loop(jnp.int32(0), jnp.int32(_GW), step=jnp.int32(_L))
                def _(j, a=a):
                    v = idx_v[jnp.int32(a), pl.ds(j, _L)]
                    hi = lax.shift_right_logical(v, jnp.int32(10))
                    lo = lax.bitwise_and(v, jnp.int32(1023))
                    m = jnp.int32(_M)
                    h0_v[a, pl.ds(j, _L)] = lax.rem(
                        hi * jnp.int32(_C0) + lo * jnp.int32(_P0), m)
                    h1_v[a, pl.ds(j, _L)] = lax.rem(
                        hi * jnp.int32(_C1) + lo * jnp.int32(_P1), m)
            for a in range(_KG):
                pltpu.async_copy(
                    table_hbm.at[h0_v.at[jnp.int32(a)]],
                    r0_v.at[pl.ds(jnp.int32(a * _GW), _GW)], sg)
                pltpu.async_copy(
                    table_hbm.at[h1_v.at[jnp.int32(a)]],
                    r1_v.at[pl.ds(jnp.int32(a * _GW), _GW)], sg)

        def drain(h0_v, h1_v, r0_v, r1_v, sg):
            for a in range(_KG):
                pltpu.make_async_copy(
                    table_hbm.at[h0_v.at[jnp.int32(a)]],
                    r0_v.at[pl.ds(jnp.int32(a * _GW), _GW)], sg).wait()
                pltpu.make_async_copy(
                    table_hbm.at[h1_v.at[jnp.int32(a)]],
                    r1_v.at[pl.ds(jnp.int32(a * _GW), _GW)], sg).wait()

        def wait_out(o_v, so):
            pltpu.make_async_copy(
                o_v, out_hbm.at[pl.ds(jnp.int32(0), _CHUNK)], so).wait()

        def add_store(g, r0_v, r1_v, o_v, so):
            @pl.loop(jnp.int32(0), jnp.int32(_CHUNK), step=jnp.int32(8))
            def _(i):
                for r in range(8):
                    row = i + jnp.int32(r)
                    for c in range(0, _D, _L):
                        o_v[row, pl.ds(jnp.int32(c), _L)] = (
                            r0_v[row, pl.ds(jnp.int32(c), _L)]
                            + r1_v[row, pl.ds(jnp.int32(c), _L)]
                        )
            out_off = (base + g * jnp.int32(rpc)) * jnp.int32(_GW)
            pltpu.async_copy(o_v, out_hbm.at[pl.ds(out_off, _CHUNK)], so)

        fire(jnp.int32(0), idx_a, h0_a, h1_a, r0_a, r1_a, sga)

        @pl.loop(jnp.int32(0), jnp.int32(pairs))
        def _(p):
            g0 = p * jnp.int32(2)
            fire(g0 + jnp.int32(1), idx_b, h0_b, h1_b, r0_b, r1_b, sgb)

            drain(h0_a, h1_a, r0_a, r1_a, sga)

            @pl.when(p > jnp.int32(0))
            def _():
                wait_out(o_a, soa)
            add_store(g0, r0_a, r1_a, o_a, soa)

            @pl.when(p < jnp.int32(pairs - 1))
            def _():
                fire(g0 + jnp.int32(2), idx_a, h0_a, h1_a, r0_a, r1_a, sga)

            drain(h0_b, h1_b, r0_b, r1_b, sgb)

            @pl.when(p > jnp.int32(0))
            def _():
                wait_out(o_b, sob)
            add_store(g0 + jnp.int32(1), r0_b, r1_b, o_b, sob)

        wait_out(o_a, soa)
        wait_out(o_b, sob)

    return k(idx128, table)


def kernel(indices, table):
    b, s = indices.shape
    half = b // 2
    n = half * s
    parts = []
    for lo in (0, half):
        idx128 = (indices[lo:lo + half].astype(jnp.int32)
                  .reshape(n // 128, 128))
        parts.append(_sc_lookup(idx128, table, n).reshape(half, s, _D))
    return jnp.concatenate(parts, axis=0)


# speedup vs baseline: 1.7454x; 1.0565x over previous
import functools

import jax
import jax.numpy as jnp
from jax import lax
from jax.experimental import pallas as pl
from jax.experimental.pallas import tpu as pltpu
from jax.experimental.pallas import tpu_sc as plsc

_PRIMES = (179424941, 179425457)
_M = 200000
_D = 64

_NC, _NS, _L = 2, 16, 16
_NW = _NC * _NS

_P0 = _PRIMES[0] % _M
_P1 = _PRIMES[1] % _M
_C0 = (1024 * _PRIMES[0]) % _M
_C1 = (1024 * _PRIMES[1]) % _M

_GW = 128
_KG = 2
_CHUNK = _GW * _KG


@functools.partial(jax.jit, static_argnums=(2,))
def _sc_lookup(idx128, table, n):
    per_w = n // _NW
    n_chunk = per_w // _CHUNK
    pairs = n_chunk // 2
    rpc = _CHUNK // _GW
    mesh = plsc.VectorSubcoreMesh(core_axis_name="c", subcore_axis_name="s")

    buf = lambda: [
        pltpu.VMEM((_KG, _GW), jnp.int32),
        pltpu.VMEM((_KG, _GW), jnp.int32),
        pltpu.VMEM((_KG, _GW), jnp.int32),
        pltpu.VMEM((_CHUNK, _D), jnp.float32),
        pltpu.VMEM((_CHUNK, _D), jnp.float32),
        pltpu.VMEM((_CHUNK, _D), jnp.float32),
        pltpu.SemaphoreType.DMA,
        pltpu.SemaphoreType.DMA,
    ]

    @functools.partial(
        pl.kernel,
        out_type=jax.ShapeDtypeStruct((n, _D), jnp.float32),
        mesh=mesh,
        compiler_params=pltpu.CompilerParams(use_tc_tiling_on_sc=False),
        scratch_types=buf() + buf(),
    )
    def k(idx_hbm, table_hbm, out_hbm,
          idx_a, h0_a, h1_a, r0_a, r1_a, o_a, sga, soa,
          idx_b, h0_b, h1_b, r0_b, r1_b, o_b, sgb, sob):
        wid = lax.axis_index("s") * jnp.int32(_NC) + lax.axis_index("c")
        base = wid * jnp.int32(per_w // _GW)

        def fire(g, idx_v, h0_v, h1_v, r0_v, r1_v, sg):
            row_off = base + g * jnp.int32(rpc)
            pltpu.sync_copy(idx_hbm.at[pl.ds(row_off, rpc)], idx_v)
            for a in range(_KG):
                @pl.loop(jnp.int32(0), jnp.int32(_GW), step=jnp.int32(_L))
                def _(j, a=a):
                    v = idx_v[jnp.int32(a), pl.ds(j, _L)]
                    hi = lax.shift_right_logical(v, jnp.int32(10))
                    lo = lax.bitwise_and(v, jnp.int32(1023))
                    m = jnp.int32(_M)
                    h0_v[a, pl.ds(j, _L)] = lax.rem(
                        hi * jnp.int32(_C0) + lo * jnp.int32(_P0), m)
                    h1_v[a, pl.ds(j, _L)] = lax.rem(
                        hi * jnp.int32(_C1) + lo * jnp.int32(_P1), m)
            for a in range(_KG):
                pltpu.async_copy(
                    table_hbm.at[h0_v.at[jnp.int32(a)]],
                    r0_v.at[pl.ds(jnp.int32(a * _GW), _GW)], sg)
                pltpu.async_copy(
                    table_hbm.at[h1_v.at[jnp.int32(a)]],
                    r1_v.at[pl.ds(jnp.int32(a * _GW), _GW)], sg)

        def drain(h0_v, h1_v, r0_v, r1_v, sg):
            for a in range(_KG):
                pltpu.make_async_copy(
                    table_hbm.at[h0_v.at[jnp.int32(a)]],
                    r0_v.at[pl.ds(jnp.int32(a * _GW), _GW)], sg).wait()
                pltpu.make_async_copy(
                    table_hbm.at[h1_v.at[jnp.int32(a)]],
                    r1_v.at[pl.ds(jnp.int32(a * _GW), _GW)], sg).wait()

        def wait_out(o_v, so):
            pltpu.make_async_copy(
                o_v, out_hbm.at[pl.ds(jnp.int32(0), _CHUNK)], so).wait()

        def add_store(g, r0_v, r1_v, o_v, so):
            @pl.loop(jnp.int32(0), jnp.int32(_CHUNK), step=jnp.int32(8))
            def _(i):
                for r in range(8):
                    row = i + jnp.int32(r)
                    for c in range(0, _D, _L):
                        o_v[row, pl.ds(jnp.int32(c), _L)] = (
                            r0_v[row, pl.ds(jnp.int32(c), _L)]
                            + r1_v[row, pl.ds(jnp.int32(c), _L)]
                        )
            out_off = (base + g * jnp.int32(rpc)) * jnp.int32(_GW)
            pltpu.async_copy(o_v, out_hbm.at[pl.ds(out_off, _CHUNK)], so)

        fire(jnp.int32(0), idx_a, h0_a, h1_a, r0_a, r1_a, sga)

        last_pair = pairs - 1 if n_chunk % 2 == 0 else pairs

        @pl.loop(jnp.int32(0), jnp.int32(pairs))
        def _(p):
            g0 = p * jnp.int32(2)
            fire(g0 + jnp.int32(1), idx_b, h0_b, h1_b, r0_b, r1_b, sgb)

            drain(h0_a, h1_a, r0_a, r1_a, sga)

            @pl.when(p > jnp.int32(0))
            def _():
                wait_out(o_a, soa)
            add_store(g0, r0_a, r1_a, o_a, soa)

            @pl.when(p < jnp.int32(last_pair))
            def _():
                fire(g0 + jnp.int32(2), idx_a, h0_a, h1_a, r0_a, r1_a, sga)

            drain(h0_b, h1_b, r0_b, r1_b, sgb)

            @pl.when(p > jnp.int32(0))
            def _():
                wait_out(o_b, sob)
            add_store(g0 + jnp.int32(1), r0_b, r1_b, o_b, sob)

        if n_chunk % 2:
            drain(h0_a, h1_a, r0_a, r1_a, sga)
            wait_out(o_a, soa)
            add_store(jnp.int32(n_chunk - 1), r0_a, r1_a, o_a, soa)

        wait_out(o_a, soa)
        wait_out(o_b, sob)

    return k(idx128, table)


def kernel(indices, table):
    b, s = indices.shape
    nsplit = 4
    part = b // nsplit
    n = part * s
    parts = []
    for i in range(nsplit):
        idx128 = (indices[i * part:(i + 1) * part].astype(jnp.int32)
                  .reshape(n // 128, 128))
        parts.append(_sc_lookup(idx128, table, n).reshape(part, s, _D))
    return jnp.concatenate(parts, axis=0)
